# baseline (speedup 1.0000x reference)
import functools
import jax
import jax.numpy as jnp
from jax import lax
from jax.experimental import pallas as pl
from jax.experimental.pallas import tpu as pltpu
from jax.experimental.pallas import tpu_sc as plsc

_CONF_THRESH = 0.25
_IOU_THRESH = 0.45
_MAX_DET = 100
_NC = 80
_ROWS = 160
_LANES = 128
_NPAD = _ROWS * _LANES
_L = 16
_NSLICE = _NPAD // _L


def _decode_body(p_ref, o_ref):
    p = p_ref[0]
    obj = p[4]
    mc = p[5] * obj
    for c in range(1, _NC):
        mc = jnp.maximum(mc, p[5 + c] * obj)
    carg = jnp.zeros((_ROWS, _LANES), jnp.int32)
    for c in range(_NC - 1, -1, -1):
        carg = jnp.where(p[5 + c] * obj == mc, c, carg)
    scores = jnp.where(mc > _CONF_THRESH, mc, -1.0)
    x = p[0]
    y = p[1]
    w = p[2]
    h = p[3]
    off = carg.astype(jnp.float32) * 4096.0
    o_ref[0, 0] = scores
    o_ref[0, 1] = (x - w / 2.0) + off
    o_ref[0, 2] = (y - h / 2.0) + off
    o_ref[0, 3] = (x + w / 2.0) + off
    o_ref[0, 4] = (y + h / 2.0) + off


def _decode(predT):
    b, c = predT.shape[0], predT.shape[1]
    return pl.pallas_call(
        _decode_body,
        grid=(b,),
        in_specs=[pl.BlockSpec((1, c, _ROWS, _LANES), lambda i: (i, 0, 0, 0))],
        out_specs=pl.BlockSpec((1, 5, _ROWS, _LANES), lambda i: (i, 0, 0, 0)),
        out_shape=jax.ShapeDtypeStruct((b, 5, _ROWS, _LANES), jnp.float32),
    )(predT)


def _sc_nms(dec_flat, b):
    mesh = plsc.VectorSubcoreMesh(core_axis_name="c", subcore_axis_name="s")
    outlen = _MAX_DET * _L

    @functools.partial(
        pl.kernel,
        mesh=mesh,
        out_type=jax.ShapeDtypeStruct((b * outlen,), jnp.float32),
        scratch_types=[
            pltpu.VMEM((_NPAD,), jnp.float32),
            pltpu.VMEM((_NPAD,), jnp.float32),
            pltpu.VMEM((_NPAD,), jnp.float32),
            pltpu.VMEM((_NPAD,), jnp.float32),
            pltpu.VMEM((_NPAD,), jnp.float32),
            pltpu.VMEM((_L,), jnp.float32),
            pltpu.VMEM((outlen,), jnp.float32),
        ],
    )
    def k(dec_hbm, out_hbm, sc_v, bx1_v, by1_v, bx2_v, by2_v, ex_v, out_v):
        wid = lax.axis_index("s") * 2 + lax.axis_index("c")

        @pl.when(wid < b)
        def _():
            base_in = wid * 5 * _NPAD
            pltpu.sync_copy(dec_hbm.at[pl.ds(base_in + 0 * _NPAD, _NPAD)], sc_v)
            pltpu.sync_copy(dec_hbm.at[pl.ds(base_in + 1 * _NPAD, _NPAD)], bx1_v)
            pltpu.sync_copy(dec_hbm.at[pl.ds(base_in + 2 * _NPAD, _NPAD)], by1_v)
            pltpu.sync_copy(dec_hbm.at[pl.ds(base_in + 3 * _NPAD, _NPAD)], bx2_v)
            pltpu.sync_copy(dec_hbm.at[pl.ds(base_in + 4 * _NPAD, _NPAD)], by2_v)

            liota = lax.iota(jnp.int32, _L)

            def perm_xor(v, s):
                return v.at[liota ^ s].get(mode="promise_in_bounds")

            def splat(v, lane):
                return v.at[liota * 0 + lane].get(mode="promise_in_bounds")

            def one_det(i, _):
                def amax_body(j, carry):
                    m16, i16 = carry
                    v = sc_v[pl.ds(j * _L, _L)]
                    gt = v > m16
                    i16 = jnp.where(gt, j * _L + liota, i16)
                    m16 = jnp.where(gt, v, m16)
                    return m16, i16

                m16, i16 = lax.fori_loop(
                    0, _NSLICE, amax_body,
                    (jnp.full((_L,), -2.0, jnp.float32),
                     jnp.zeros((_L,), jnp.int32)))
                mg = m16
                for s in (1, 2, 4, 8):
                    mg = jnp.maximum(mg, perm_xor(mg, s))
                cand = jnp.where(m16 == mg, i16, _NPAD)
                for s in (1, 2, 4, 8):
                    cand = jnp.minimum(cand, perm_xor(cand, s))
                idx16 = cand

                gidx = base_in + (jnp.minimum(liota, 3) + 1) * _NPAD + idx16
                pltpu.sync_copy(dec_hbm.at[gidx], ex_v)
                ev = ex_v[...]
                ebx1 = splat(ev, 0)
                eby1 = splat(ev, 1)
                ebx2 = splat(ev, 2)
                eby2 = splat(ev, 3)
                a1 = (ebx2 - ebx1) * (eby2 - eby1)
                valid = mg > 0.0
                vf = jnp.where(valid, 1.0, 0.0)

                ecls = (ebx1 * (1.0 / 4096.0) + 0.5).astype(jnp.int32)
                eclsf = ecls.astype(jnp.float32)
                eoff = eclsf * 4096.0
                row = (
                    jnp.where(liota == 0, ebx1 - eoff, 0.0)
                    + jnp.where(liota == 1, eby1 - eoff, 0.0)
                    + jnp.where(liota == 2, ebx2 - eoff, 0.0)
                    + jnp.where(liota == 3, eby2 - eoff, 0.0)
                    + jnp.where(liota == 4, jnp.maximum(mg, 0.0), 0.0)
                    + jnp.where(liota == 5, eclsf, 0.0)
                ) * vf
                out_v[pl.ds(i * _L, _L)] = row

                def sup_body(j, _):
                    s = pl.ds(j * _L, _L)
                    bx1 = bx1_v[s]
                    by1 = by1_v[s]
                    bx2 = bx2_v[s]
                    by2 = by2_v[s]
                    xx1 = jnp.maximum(ebx1, bx1)
                    yy1 = jnp.maximum(eby1, by1)
                    xx2 = jnp.minimum(ebx2, bx2)
                    yy2 = jnp.minimum(eby2, by2)
                    inter = jnp.maximum(xx2 - xx1, 0.0) * jnp.maximum(
                        yy2 - yy1, 0.0)
                    a2 = (bx2 - bx1) * (by2 - by1)
                    iou = inter / (a1 + a2 - inter + 1e-9)
                    sup = (iou > _IOU_THRESH) & valid
                    kill = sup | (j * _L + liota == idx16)
                    sc = sc_v[s]
                    sc_v[s] = jnp.where(kill, -1.0, sc)
                    return 0

                lax.fori_loop(0, _NSLICE, sup_body, 0)
                return 0

            lax.fori_loop(0, _MAX_DET, one_det, 0)
            pltpu.sync_copy(out_v, out_hbm.at[pl.ds(wid * outlen, outlen)])

    return k(dec_flat)


def kernel(pred):
    b, n, c = pred.shape
    predT = jnp.swapaxes(pred, 1, 2)
    predT = jnp.pad(predT, ((0, 0), (0, 0), (0, _NPAD - n)))
    predT = predT.reshape(b, c, _ROWS, _LANES)
    dec = _decode(predT).reshape(b * 5 * _NPAD)
    out = _sc_nms(dec, b)
    out = out.reshape(b, _MAX_DET, _L)
    return out[:, :, :6]
